# baseline (speedup 1.0000x reference)
import functools

import jax
import jax.numpy as jnp
from jax import lax
from jax.experimental import pallas as pl
from jax.experimental.pallas import tpu as pltpu
from jax.experimental.pallas import tpu_sc as plsc

_PAD = 0


def _pos_body(inp_ref, pos_ref):
    x = inp_ref[...]
    mask = x != _PAD
    mf = mask.astype(jnp.bfloat16)
    s = x.shape[1]
    r = lax.broadcasted_iota(jnp.int32, (s, s), 0)
    c = lax.broadcasted_iota(jnp.int32, (s, s), 1)
    tri = (r <= c).astype(jnp.bfloat16)
    pos_f = jnp.dot(mf, tri, preferred_element_type=jnp.float32)
    pos = pos_f.astype(jnp.int32)
    pos_ref[...] = jnp.where(mask, pos, _PAD)


def _positions(inp):
    b, s = inp.shape
    blk = 256
    return pl.pallas_call(
        _pos_body,
        out_shape=jax.ShapeDtypeStruct((b, s), jnp.int32),
        grid=(b // blk,),
        in_specs=[pl.BlockSpec((blk, s), lambda i: (i, 0))],
        out_specs=pl.BlockSpec((blk, s), lambda i: (i, 0)),
    )(inp)


_NBUF = 4


def _make_gather(n, v, d):
    nw = 32
    k = 128
    per_w = n // nw
    n_chunks = per_w // k
    ng = n_chunks // _NBUF
    assert per_w % k == 0 and n_chunks % _NBUF == 0

    mesh = plsc.VectorSubcoreMesh(core_axis_name="c", subcore_axis_name="s")

    @functools.partial(
        pl.kernel,
        mesh=mesh,
        out_type=jax.ShapeDtypeStruct((n, d), jnp.float32),
        scratch_types=[
            pltpu.VMEM((n_chunks, k), jnp.int32),
            pltpu.VMEM((_NBUF, k, d), jnp.float32),
            pltpu.VMEM_SHARED((v, d), jnp.float32),
            pltpu.SemaphoreType.DMA,
            pltpu.SemaphoreType.DMA,
            pltpu.SemaphoreType.DMA,
            pltpu.SemaphoreType.DMA,
            pltpu.SemaphoreType.DMA,
            pltpu.SemaphoreType.DMA,
            pltpu.SemaphoreType.DMA,
            pltpu.SemaphoreType.DMA,
            pltpu.SemaphoreType.DMA,
        ],
    )
    def gather(pos_hbm, table_hbm, out_hbm, idx_v, rows_v, table_sh,
               sl, *sems):
        sg = sems[:_NBUF]
        ss = sems[_NBUF:]
        sid = lax.axis_index("s")
        wid = sid * 2 + lax.axis_index("c")
        cbase = wid * n_chunks

        @pl.when(sid == 0)
        def _():
            pltpu.sync_copy(table_hbm, table_sh)

        pltpu.async_copy(pos_hbm.at[pl.ds(cbase, n_chunks)], idx_v, sl)
        pltpu.make_async_copy(pos_hbm.at[pl.ds(cbase, n_chunks)], idx_v,
                              sl).wait()
        plsc.subcore_barrier()

        def out_slice(c):
            return out_hbm.at[pl.ds((cbase + c) * k, k)]

        def body(j, carry):
            c = j * _NBUF

            @pl.when(j > 0)
            def _():
                for b in range(_NBUF):
                    pltpu.make_async_copy(rows_v.at[b], out_slice(c - _NBUF + b),
                                          ss[b]).wait()

            for b in range(_NBUF):
                pltpu.async_copy(table_sh.at[idx_v.at[c + b]],
                                 rows_v.at[b], sg[b])
            for b in range(_NBUF):
                pltpu.make_async_copy(table_sh.at[idx_v.at[c + b]],
                                      rows_v.at[b], sg[b]).wait()
                pltpu.async_copy(rows_v.at[b], out_slice(c + b), ss[b])
            return carry

        lax.fori_loop(0, ng, body, 0)
        for b in range(_NBUF):
            pltpu.make_async_copy(rows_v.at[b], out_slice(b), ss[b]).wait()

    return gather


def kernel(input, table):
    b, s = input.shape
    v, d = table.shape
    inp = input.astype(jnp.int32)
    pos = _positions(inp)
    n = b * s
    out = _make_gather(n, v, d)(pos.reshape(n // 128, 128), table)
    return out.reshape(b, s, d)


# speedup vs baseline: 1.2522x; 1.2522x over previous
import functools

import jax
import jax.numpy as jnp
from jax import lax
from jax.experimental import pallas as pl
from jax.experimental.pallas import tpu as pltpu
from jax.experimental.pallas import tpu_sc as plsc

_PAD = 0


def _pos_body(inp_ref, pos_ref):
    x = inp_ref[...]
    mask = x != _PAD
    mf = mask.astype(jnp.bfloat16)
    s = x.shape[1]
    r = lax.broadcasted_iota(jnp.int32, (s, s), 0)
    c = lax.broadcasted_iota(jnp.int32, (s, s), 1)
    tri = (r <= c).astype(jnp.bfloat16)
    pos_f = jnp.dot(mf, tri, preferred_element_type=jnp.float32)
    pos = pos_f.astype(jnp.int32)
    pos_ref[...] = jnp.where(mask, pos, _PAD)


def _positions(inp):
    b, s = inp.shape
    blk = b
    return pl.pallas_call(
        _pos_body,
        out_shape=jax.ShapeDtypeStruct((b, s), jnp.int32),
        grid=(b // blk,),
        in_specs=[pl.BlockSpec((blk, s), lambda i: (i, 0))],
        out_specs=pl.BlockSpec((blk, s), lambda i: (i, 0)),
    )(inp)


_NBUF = 5


def _make_gather(n, v, d):
    nw = 32
    k = 128
    per_w = n // nw
    n_chunks = per_w // k
    ng = n_chunks // _NBUF
    assert per_w % k == 0 and n_chunks % _NBUF == 0

    mesh = plsc.VectorSubcoreMesh(core_axis_name="c", subcore_axis_name="s")

    @functools.partial(
        pl.kernel,
        mesh=mesh,
        out_type=jax.ShapeDtypeStruct((n, d), jnp.float32),
        scratch_types=[
            pltpu.VMEM((n_chunks, k), jnp.int32),
            pltpu.VMEM((_NBUF, k, d), jnp.float32),
            pltpu.VMEM_SHARED((v, d), jnp.float32),
        ] + [pltpu.SemaphoreType.DMA] * (1 + 2 * _NBUF),
    )
    def gather(pos_hbm, table_hbm, out_hbm, idx_v, rows_v, table_sh,
               sl, *sems):
        sg = sems[:_NBUF]
        ss = sems[_NBUF:]
        sid = lax.axis_index("s")
        wid = sid * 2 + lax.axis_index("c")
        cbase = wid * n_chunks

        @pl.when(sid == 0)
        def _():
            pltpu.sync_copy(table_hbm, table_sh)

        pltpu.async_copy(pos_hbm.at[pl.ds(cbase, n_chunks)], idx_v, sl)
        pltpu.make_async_copy(pos_hbm.at[pl.ds(cbase, n_chunks)], idx_v,
                              sl).wait()
        plsc.subcore_barrier()

        def out_slice(c):
            return out_hbm.at[pl.ds((cbase + c) * k, k)]

        def body(j, carry):
            c = j * _NBUF

            @pl.when(j > 0)
            def _():
                for b in range(_NBUF):
                    pltpu.make_async_copy(rows_v.at[b], out_slice(c - _NBUF + b),
                                          ss[b]).wait()
                    pltpu.async_copy(table_sh.at[idx_v.at[c + b]],
                                     rows_v.at[b], sg[b])

            @pl.when(j == 0)
            def _():
                for b in range(_NBUF):
                    pltpu.async_copy(table_sh.at[idx_v.at[c + b]],
                                     rows_v.at[b], sg[b])

            for b in range(_NBUF):
                pltpu.make_async_copy(table_sh.at[idx_v.at[c + b]],
                                      rows_v.at[b], sg[b]).wait()
                pltpu.async_copy(rows_v.at[b], out_slice(c + b), ss[b])
            return carry

        lax.fori_loop(0, ng, body, 0)
        for b in range(_NBUF):
            pltpu.make_async_copy(rows_v.at[b], out_slice(b), ss[b]).wait()

    return gather


def kernel(input, table):
    b, s = input.shape
    v, d = table.shape
    inp = input.astype(jnp.int32)
    pos = _positions(inp)
    n = b * s
    out = _make_gather(n, v, d)(pos.reshape(n // 128, 128), table)
    return out.reshape(b, s, d)
